# Initial kernel scaffold; baseline (speedup 1.0000x reference)
import jax
import jax.numpy as jnp
from jax.experimental import pallas as pl


def kernel(x, W):
    raise NotImplementedError("write your pallas kernel here")


# speedup vs baseline: 1.2126x; 1.2126x over previous
import functools

import jax
import jax.numpy as jnp
from jax import lax
from jax.experimental import pallas as pl

EPS = 1e-9
CAPACITY_FACTOR = 1.25
MIN_CAPACITY = 4


def _phase1_body(x_ref, w_ref, meta_ref, *, cap, num_gates, group_size):
    xb = x_ref[0]
    w = w_ref[...]
    logits = lax.dot_general(
        xb, w, (((1,), (1,)), ((), ())), preferred_element_type=jnp.float32
    )
    m = jnp.max(logits, axis=-1, keepdims=True)
    ex = jnp.exp(logits - m)
    raw = ex / jnp.sum(ex, axis=-1, keepdims=True)

    lane = lax.broadcasted_iota(jnp.int32, (group_size, num_gates), 1).astype(
        jnp.float32
    )
    g1 = jnp.max(raw, axis=-1, keepdims=True)
    i1 = jnp.min(jnp.where(raw >= g1, lane, jnp.float32(1e9)), axis=-1, keepdims=True)
    mask1 = (lane == i1).astype(jnp.float32)
    wo = raw * (1.0 - mask1)
    g2 = jnp.max(wo, axis=-1, keepdims=True)
    i2 = jnp.min(jnp.where(wo >= g2, lane, jnp.float32(1e9)), axis=-1, keepdims=True)
    mask2 = (lane == i2).astype(jnp.float32)

    denom = g1 + g2 + EPS
    g1n = g1 / denom
    g2n = g2 / denom

    proxy_m = jnp.mean(raw, axis=0, keepdims=True)
    dens1 = jnp.mean(mask1, axis=0, keepdims=True)
    partial = jnp.sum(proxy_m * dens1)

    r = lax.broadcasted_iota(jnp.int32, (group_size, group_size), 0)
    c = lax.broadcasted_iota(jnp.int32, (group_size, group_size), 1)
    lt = (r > c).astype(jnp.float32)
    pos1 = jnp.dot(lt, mask1, preferred_element_type=jnp.float32)
    pos1_tok = jnp.sum(pos1 * mask1, axis=-1, keepdims=True)
    keep1 = (pos1_tok < cap).astype(jnp.float32)
    cnt1 = jnp.sum(mask1 * keep1, axis=0, keepdims=True)
    pos2 = jnp.dot(lt, mask2, preferred_element_type=jnp.float32) + cnt1
    pos2_tok = jnp.sum(pos2 * mask2, axis=-1, keepdims=True)
    keep2 = (pos2_tok < cap).astype(jnp.float32)

    val1 = g1n * keep1
    val2 = g2n * keep2
    idx1 = jnp.where(keep1 > 0.0, i1 * cap + pos1_tok, jnp.float32(-1.0))
    idx2 = jnp.where(keep2 > 0.0, i2 * cap + pos2_tok, jnp.float32(-1.0))
    losscol = jnp.zeros((group_size, 1), jnp.float32) + partial
    pad = jnp.zeros((group_size, 1), jnp.float32)
    meta_ref[0] = jnp.concatenate(
        [idx1, val1, idx2, val2, losscol, pad, pad, pad], axis=1
    )


def _phase2_body(meta_ref, comb_ref, disp_ref, *, num_cols, ts):
    meta = meta_ref[0]
    i1 = meta[:, 0:1]
    v1 = meta[:, 1:2]
    i2 = meta[:, 2:3]
    v2 = meta[:, 3:4]
    col = lax.broadcasted_iota(jnp.int32, (ts, num_cols), 1).astype(jnp.float32)
    m1 = col == i1
    m2 = col == i2
    comb_ref[0] = jnp.where(m1, v1, 0.0) + jnp.where(m2, v2, 0.0)
    disp_ref[0] = jnp.where(m1, 1.0, 0.0) + jnp.where(m2, 1.0, 0.0)


def kernel(x, W):
    b, s, d = x.shape
    e = W.shape[0]
    cap = max(min(s, int(s * CAPACITY_FACTOR / e)), MIN_CAPACITY)
    nc = e * cap

    meta = pl.pallas_call(
        functools.partial(
            _phase1_body, cap=float(cap), num_gates=e, group_size=s
        ),
        grid=(b,),
        in_specs=[
            pl.BlockSpec((1, s, d), lambda i: (i, 0, 0)),
            pl.BlockSpec((e, d), lambda i: (0, 0)),
        ],
        out_specs=pl.BlockSpec((1, s, 8), lambda i: (i, 0, 0)),
        out_shape=jax.ShapeDtypeStruct((b, s, 8), jnp.float32),
    )(x, W)

    ts = 256
    comb2, disp2 = pl.pallas_call(
        functools.partial(_phase2_body, num_cols=nc, ts=ts),
        grid=(b, s // ts),
        in_specs=[pl.BlockSpec((1, ts, 8), lambda i, j: (i, j, 0))],
        out_specs=[
            pl.BlockSpec((1, ts, nc), lambda i, j: (i, j, 0)),
            pl.BlockSpec((1, ts, nc), lambda i, j: (i, j, 0)),
        ],
        out_shape=[
            jax.ShapeDtypeStruct((b, s, nc), jnp.float32),
            jax.ShapeDtypeStruct((b, s, nc), jnp.float32),
        ],
    )(meta)

    dispatch = disp2.reshape(b, s, e, cap)
    combine = comb2.reshape(b, s, e, cap)
    loss = jnp.sum(meta[:, 0, 4]) * (float(e) / float(b))
    return (dispatch, combine, loss)
